# baseline (speedup 1.0000x reference)
import functools

import jax
import jax.numpy as jnp
from jax import lax
from jax.experimental import pallas as pl
from jax.experimental.pallas import tpu as pltpu
from jax.experimental.pallas import tpu_sc as plsc

D = 128
L = 16
NC = 2
NS = 16
NW = NC * NS
CHUNK = 128


def _sc_embed(cond2d, cat2d, cond_table, cat_table, cat_start_pad, n_chunks):
    B = NW * n_chunks * CHUNK
    n_cs = cat_start_pad.shape[0]
    mesh = plsc.VectorSubcoreMesh(core_axis_name="c", subcore_axis_name="s")

    @functools.partial(
        pl.kernel,
        out_type=jax.ShapeDtypeStruct((B, D), jnp.float32),
        mesh=mesh,
        compiler_params=pltpu.CompilerParams(needs_layout_passes=False),
        scratch_types=[
            pltpu.VMEM((n_cs,), jnp.int32),
            pltpu.VMEM((n_chunks, CHUNK), jnp.int32),
            pltpu.VMEM((n_chunks, CHUNK), jnp.int32),
            pltpu.VMEM((2, CHUNK, D), jnp.float32),
            pltpu.VMEM((2, CHUNK, D), jnp.float32),
            pltpu.SemaphoreType.DMA,
            pltpu.SemaphoreType.DMA,
            pltpu.SemaphoreType.DMA,
            pltpu.SemaphoreType.DMA,
        ],
    )
    def k(cond_hbm, cat_hbm, condtab_hbm, cattab_hbm, cs_hbm, out_hbm,
          cs_v, cond_v, full_v, rcond, rcat, semc0, semc1, semd0, semd1):
        wid = lax.axis_index("s") * NC + lax.axis_index("c")
        crow = wid * n_chunks
        base = wid * n_chunks * CHUNK
        sems = ((semc0, semd0), (semc1, semd1))

        pltpu.sync_copy(cs_hbm, cs_v)
        pltpu.sync_copy(cond_hbm.at[wid], cond_v)
        pltpu.sync_copy(cat_hbm.at[wid], full_v)

        @pl.loop(0, n_chunks)
        def _(g):
            for kk in range(CHUNK // L):
                sl = pl.ds(kk * L, L)
                starts = plsc.load_gather(cs_v, [cond_v[g, sl]])
                full_v[g, sl] = starts + full_v[g, sl]

        def fire(g, b):
            semc, semd = sems[b]
            pltpu.async_copy(cattab_hbm.at[full_v.at[g]], rcat.at[b], semc)
            pltpu.async_copy(condtab_hbm.at[cond_v.at[g]], rcond.at[b], semd)

        def drain(b):
            semc, semd = sems[b]
            pltpu.make_async_copy(
                cattab_hbm.at[pl.ds(0, CHUNK)], rcat.at[b], semc).wait()
            pltpu.make_async_copy(
                condtab_hbm.at[pl.ds(0, CHUNK)], rcond.at[b], semd).wait()

        def consume(g, b):
            drain(b)

            @pl.loop(0, CHUNK)
            def _(r):
                for kk in range(D // L):
                    sl = pl.ds(kk * L, L)
                    plsc.addupdate(rcat.at[b, r, sl], rcond[b, r, sl])

            pltpu.sync_copy(rcat.at[b], out_hbm.at[pl.ds(base + g * CHUNK, CHUNK)])

        fire(0, 0)

        @pl.loop(0, n_chunks, step=2)
        def _(g):
            fire(g + 1, 1)
            consume(g, 0)

            @pl.when(g + 2 < n_chunks)
            def _():
                fire(g + 2, 0)

            consume(g + 1, 1)

    return k(cond2d, cat2d, cond_table, cat_table, cat_start_pad)


def kernel(cond_ids, cat_ids, cond_table, cat_table, cat_start):
    bt, f = cond_ids.shape
    b_total = bt * f
    n_chunks = b_total // (NW * CHUNK)
    cond2d = cond_ids.reshape(NW, -1, CHUNK).astype(jnp.int32)
    cat2d = cat_ids.reshape(NW, -1, CHUNK).astype(jnp.int32)
    cs = cat_start.astype(jnp.int32)
    n_pad = ((cs.shape[0] + 7) // 8) * 8
    cs_pad = jnp.zeros((n_pad,), jnp.int32).at[: cs.shape[0]].set(cs)
    out = _sc_embed(cond2d, cat2d, cond_table, cat_table, cs_pad, n_chunks)
    return out.reshape(bt, f, cond_table.shape[1])
